# baseline (speedup 1.0000x reference)
import functools

import jax
import jax.numpy as jnp
from jax import lax
from jax.experimental import pallas as pl
from jax.experimental.pallas import tpu as pltpu
from jax.experimental.pallas import tpu_sc as plsc

_N_E = 8192
_E_DIM = 256
_BETA = 0.25

_BM = 256
_BN = 1024
_M = 16384
_NB = _M // _BM
_NCHUNK = _N_E // _BN

_NW = 32
_ROWS_PER_W = _M // _NW
_SC_CHUNK = 128
_NSC_CHUNK = _ROWS_PER_W // _SC_CHUNK


_FOLD_BOUNDS = (0, 2736, 5472, 8192)
_NFOLD = len(_FOLD_BOUNDS) - 1


def _argmin_body(zsq_ref, esq_ref, z_ref, et_ref, idx_ref, loss_ref):
    i = pl.program_id(0)
    z = z_ref[...]
    zsq = zsq_ref[...]
    inf = jnp.float32(jnp.inf)
    cv = [None] * _NFOLD
    ci = [None] * _NFOLD
    for jn in range(_NCHUNK):
        et = et_ref[:, jn * _BN:(jn + 1) * _BN]
        esq = esq_ref[:, jn * _BN:(jn + 1) * _BN]
        m2 = lax.dot_general(z, et, (((1,), (0,)), ((), ())),
                             preferred_element_type=jnp.float32)
        d = (zsq + esq) + m2
        col = lax.broadcasted_iota(jnp.int32, (_BM, _BN), 1) + jn * _BN
        b0, b1 = jn * _BN, (jn + 1) * _BN
        for c in range(_NFOLD):
            lo = max(_FOLD_BOUNDS[c], b0)
            hi = min(_FOLD_BOUNDS[c + 1], b1)
            if lo >= hi:
                continue
            if lo == b0 and hi == b1:
                dm = d
            else:
                dm = jnp.where((col >= lo) & (col < hi), d, inf)
            mv = jnp.min(dm, axis=1, keepdims=True)
            cidx = jnp.min(jnp.where(dm == mv, col, _N_E), axis=1, keepdims=True)
            if cv[c] is None:
                cv[c], ci[c] = mv, cidx
            else:
                upd = mv < cv[c]
                ci[c] = jnp.where(upd, cidx, ci[c])
                cv[c] = jnp.where(upd, mv, cv[c])
    acc_v, acc_i, acc_e = cv[0], ci[0], cv[0]
    for c in range(1, _NFOLD):
        acc_v = acc_v.astype(jnp.bfloat16).astype(jnp.float32)
        upd = cv[c] < acc_v
        acc_i = jnp.where(upd, ci[c], acc_i)
        acc_e = jnp.where(upd, cv[c], acc_e)
        acc_v = jnp.where(upd, cv[c], acc_v)
    best_idx = acc_i
    best_val = acc_e
    idx_ref[...] = best_idx

    @pl.when(i == 0)
    def _init():
        loss_ref[...] = jnp.zeros((1, 1), dtype=jnp.float32)

    loss_ref[...] += jnp.sum(best_val, axis=(0, 1), keepdims=True)

    @pl.when(i == _NB - 1)
    def _finish():
        loss_ref[...] = loss_ref[...] * (_BETA / float(_M * _E_DIM))


def _tc_argmin(zsq, esq, z_flat, et):
    return pl.pallas_call(
        _argmin_body,
        grid=(_NB,),
        in_specs=[
            pl.BlockSpec((_BM, 1), lambda i: (i, 0)),
            pl.BlockSpec((1, _N_E), lambda i: (0, 0)),
            pl.BlockSpec((_BM, _E_DIM), lambda i: (i, 0)),
            pl.BlockSpec((_E_DIM, _N_E), lambda i: (0, 0)),
        ],
        out_specs=[
            pl.BlockSpec((_BM, 1), lambda i: (i, 0)),
            pl.BlockSpec((1, 1), lambda i: (0, 0)),
        ],
        out_shape=[
            jax.ShapeDtypeStruct((_M, 1), jnp.int32),
            jax.ShapeDtypeStruct((1, 1), jnp.float32),
        ],
    )(zsq, esq, z_flat, et)


def _sc_body(e_hbm, idx_hbm, out_hbm, idx_v, rows_v, sem):
    wid = lax.axis_index("s") * 2 + lax.axis_index("c")
    for c in range(_NSC_CHUNK):
        base = wid * _ROWS_PER_W + c * _SC_CHUNK
        pltpu.sync_copy(idx_hbm.at[pl.ds(base, _SC_CHUNK)], idx_v)
        pltpu.async_copy(e_hbm.at[idx_v], rows_v, sem).wait()
        pltpu.sync_copy(rows_v, out_hbm.at[pl.ds(base, _SC_CHUNK)])


def _sc_gather(embeddings, indices):
    mesh = plsc.VectorSubcoreMesh(core_axis_name="c", subcore_axis_name="s")
    return pl.kernel(
        _sc_body,
        out_type=jax.ShapeDtypeStruct((_M, _E_DIM), jnp.float32),
        mesh=mesh,
        scratch_types=[
            pltpu.VMEM((_SC_CHUNK,), jnp.int32),
            pltpu.VMEM((_SC_CHUNK, _E_DIM), jnp.float32),
            pltpu.SemaphoreType.DMA,
        ],
    )(embeddings, indices)


def kernel(z, embeddings):
    old_shape = z.shape
    z_flat = z.reshape(-1, _E_DIM)
    zsq = jnp.sum(z_flat ** 2, axis=1, keepdims=True)
    esq = jnp.sum(embeddings ** 2, axis=1).reshape(1, _N_E)
    z_bf = z_flat.astype(jnp.bfloat16)
    et_bf = (-2.0 * embeddings).T.astype(jnp.bfloat16)
    idx2, loss2 = _tc_argmin(zsq, esq, z_bf, et_bf)
    indices = idx2.reshape(_M)
    z_q = _sc_gather(embeddings, indices)
    return z_q.reshape(old_shape), indices, loss2[0, 0]


# speedup vs baseline: 1.1573x; 1.1573x over previous
import functools

import jax
import jax.numpy as jnp
from jax import lax
from jax.experimental import pallas as pl
from jax.experimental.pallas import tpu as pltpu
from jax.experimental.pallas import tpu_sc as plsc

_N_E = 8192
_E_DIM = 256
_BETA = 0.25

_BM = 1024
_BN = 1024
_M = 16384
_NB = _M // _BM
_NCHUNK = _N_E // _BN

_NW = 32
_ROWS_PER_W = _M // _NW
_SC_CHUNK = 128
_NSC_CHUNK = _ROWS_PER_W // _SC_CHUNK


_FOLD_BOUNDS = (0, 2736, 5472, 8192)
_NFOLD = len(_FOLD_BOUNDS) - 1


def _argmin_body(zsq_ref, esq_ref, z_ref, et_ref, idx_ref, loss_ref):
    i = pl.program_id(0)
    z = z_ref[...]
    zsq = zsq_ref[...]
    inf = jnp.float32(jnp.inf)
    cv = [None] * _NFOLD
    ci = [None] * _NFOLD
    for jn in range(_NCHUNK):
        et = et_ref[:, jn * _BN:(jn + 1) * _BN]
        esq = esq_ref[:, jn * _BN:(jn + 1) * _BN]
        m2 = lax.dot_general(z, et, (((1,), (0,)), ((), ())),
                             preferred_element_type=jnp.float32)
        d = (zsq + esq) + m2
        col = lax.broadcasted_iota(jnp.int32, (_BM, _BN), 1) + jn * _BN
        b0, b1 = jn * _BN, (jn + 1) * _BN
        for c in range(_NFOLD):
            lo = max(_FOLD_BOUNDS[c], b0)
            hi = min(_FOLD_BOUNDS[c + 1], b1)
            if lo >= hi:
                continue
            if lo == b0 and hi == b1:
                dm = d
            else:
                dm = jnp.where((col >= lo) & (col < hi), d, inf)
            mv = jnp.min(dm, axis=1, keepdims=True)
            cidx = jnp.min(jnp.where(dm == mv, col, _N_E), axis=1, keepdims=True)
            if cv[c] is None:
                cv[c], ci[c] = mv, cidx
            else:
                upd = mv < cv[c]
                ci[c] = jnp.where(upd, cidx, ci[c])
                cv[c] = jnp.where(upd, mv, cv[c])
    acc_v, acc_i, acc_e = cv[0], ci[0], cv[0]
    for c in range(1, _NFOLD):
        acc_v = acc_v.astype(jnp.bfloat16).astype(jnp.float32)
        upd = cv[c] < acc_v
        acc_i = jnp.where(upd, ci[c], acc_i)
        acc_e = jnp.where(upd, cv[c], acc_e)
        acc_v = jnp.where(upd, cv[c], acc_v)
    best_idx = acc_i
    best_val = acc_e
    idx_ref[...] = best_idx

    @pl.when(i == 0)
    def _init():
        loss_ref[...] = jnp.zeros((1, 1), dtype=jnp.float32)

    loss_ref[...] += jnp.sum(best_val, axis=(0, 1), keepdims=True)

    @pl.when(i == _NB - 1)
    def _finish():
        loss_ref[...] = loss_ref[...] * (_BETA / float(_M * _E_DIM))


def _tc_argmin(zsq, esq, z_flat, et):
    return pl.pallas_call(
        _argmin_body,
        grid=(_NB,),
        in_specs=[
            pl.BlockSpec((_BM, 1), lambda i: (i, 0)),
            pl.BlockSpec((1, _N_E), lambda i: (0, 0)),
            pl.BlockSpec((_BM, _E_DIM), lambda i: (i, 0)),
            pl.BlockSpec((_E_DIM, _N_E), lambda i: (0, 0)),
        ],
        out_specs=[
            pl.BlockSpec((_BM, 1), lambda i: (i, 0)),
            pl.BlockSpec((1, 1), lambda i: (0, 0)),
        ],
        out_shape=[
            jax.ShapeDtypeStruct((_M, 1), jnp.int32),
            jax.ShapeDtypeStruct((1, 1), jnp.float32),
        ],
    )(zsq, esq, z_flat, et)


def _sc_body(e_hbm, idx_hbm, out_hbm, idx_v, rows_v, sem):
    wid = lax.axis_index("s") * 2 + lax.axis_index("c")
    for c in range(_NSC_CHUNK):
        base = wid * _ROWS_PER_W + c * _SC_CHUNK
        pltpu.sync_copy(idx_hbm.at[pl.ds(base, _SC_CHUNK)], idx_v)
        pltpu.async_copy(e_hbm.at[idx_v], rows_v, sem).wait()
        pltpu.sync_copy(rows_v, out_hbm.at[pl.ds(base, _SC_CHUNK)])


def _sc_gather(embeddings, indices):
    mesh = plsc.VectorSubcoreMesh(core_axis_name="c", subcore_axis_name="s")
    return pl.kernel(
        _sc_body,
        out_type=jax.ShapeDtypeStruct((_M, _E_DIM), jnp.float32),
        mesh=mesh,
        scratch_types=[
            pltpu.VMEM((_SC_CHUNK,), jnp.int32),
            pltpu.VMEM((_SC_CHUNK, _E_DIM), jnp.float32),
            pltpu.SemaphoreType.DMA,
        ],
    )(embeddings, indices)


def kernel(z, embeddings):
    old_shape = z.shape
    z_flat = z.reshape(-1, _E_DIM)
    zsq = jnp.sum(z_flat ** 2, axis=1, keepdims=True)
    esq = jnp.sum(embeddings ** 2, axis=1).reshape(1, _N_E)
    z_bf = z_flat.astype(jnp.bfloat16)
    et_bf = (-2.0 * embeddings).T.astype(jnp.bfloat16)
    idx2, loss2 = _tc_argmin(zsq, esq, z_bf, et_bf)
    indices = idx2.reshape(_M)
    z_q = _sc_gather(embeddings, indices)
    return z_q.reshape(old_shape), indices, loss2[0, 0]
